# Initial kernel scaffold; baseline (speedup 1.0000x reference)
import jax
import jax.numpy as jnp
from jax.experimental import pallas as pl


def kernel(x, edge_index, edge_attr, batch, W1, b1, W2, b2, W3, b3, Wlin, blin):
    raise NotImplementedError("write your pallas kernel here")


# speedup vs baseline: 9.0420x; 9.0420x over previous
import dataclasses
import functools

import jax
import jax.numpy as jnp
from jax import lax
from jax.experimental import pallas as pl
from jax.experimental.pallas import tpu as pltpu
from jax.experimental.pallas import tpu_sc as plsc

_NC = 2
_NS = 16
_L = 16
_D = 128


def _pick_chunk(epw):
    for k in range(128, 7, -8):
        if epw % k == 0:
            return k
    raise ValueError(f"no valid chunk size for {epw} edges per worker")


def _sc_mesh():
    return plsc.VectorSubcoreMesh(core_axis_name="c", subcore_axis_name="s")


def _sc_compiler_params():
    cp = pltpu.CompilerParams()
    if "needs_layout_passes" in pltpu.CompilerParams.__dataclass_fields__:
        cp = dataclasses.replace(cp, needs_layout_passes=False)
    return cp


def _sc_degree(dst, w, zeros_nd, npad):
    e = dst.shape[0]
    nw = _NC * _NS
    epw = e // nw
    kchunk = _pick_chunk(epw)
    nch = epw // kchunk
    rows = npad // _NS

    @functools.partial(
        pl.kernel,
        mesh=_sc_mesh(),
        compiler_params=_sc_compiler_params(),
        out_type=jax.ShapeDtypeStruct((_NC, npad, _D), jnp.float32),
        scratch_types=[
            pltpu.VMEM_SHARED((npad, _D), jnp.float32),
            pltpu.VMEM((epw,), jnp.float32),
            pltpu.VMEM((kchunk,), jnp.int32),
            pltpu.VMEM((kchunk, _D), jnp.float32),
        ],
    )
    def k(dst_hbm, w_hbm, z_hbm, out_hbm, deg_sh, w_v, dst_v, wrow_v):
        c = lax.axis_index("c")
        s = lax.axis_index("s")
        wid = s * _NC + c
        base = wid * epw
        pltpu.sync_copy(z_hbm.at[pl.ds(s * rows, rows)],
                        deg_sh.at[pl.ds(s * rows, rows)])
        pltpu.sync_copy(w_hbm.at[pl.ds(base, epw)], w_v)
        plsc.subcore_barrier()

        @pl.loop(0, nch)
        def _(ci):
            off = ci * kchunk
            pltpu.sync_copy(dst_hbm.at[pl.ds(base + off, kchunk)], dst_v)

            @pl.loop(0, kchunk)
            def _(kk):
                wspl = plsc.load_gather(
                    w_v, [jnp.full((_L,), off + kk, jnp.int32)])
                for j in range(_D // _L):
                    wrow_v[kk, pl.ds(j * _L, _L)] = wspl

            pltpu.sync_copy(wrow_v, deg_sh.at[dst_v], add=True)

        plsc.subcore_barrier()
        pltpu.sync_copy(deg_sh.at[pl.ds(s * rows, rows)],
                        out_hbm.at[c, pl.ds(s * rows, rows)])

    return k(dst, w, zeros_nd)


def _sc_aggregate(h, src, dst, w, zeros_nd):
    n = h.shape[0]
    e = src.shape[0]
    npad = zeros_nd.shape[0]
    nw = _NC * _NS
    epw = e // nw
    kchunk = _pick_chunk(epw)
    nch = epw // kchunk
    rows = npad // _NS

    @functools.partial(
        pl.kernel,
        mesh=_sc_mesh(),
        compiler_params=_sc_compiler_params(),
        out_type=jax.ShapeDtypeStruct((_NC, npad, _D), jnp.float32),
        scratch_types=[
            pltpu.VMEM_SHARED((npad, _D), jnp.float32),
            pltpu.VMEM((epw,), jnp.int32),
            pltpu.VMEM((epw,), jnp.float32),
            pltpu.VMEM((kchunk,), jnp.int32),
            pltpu.VMEM((kchunk, _D), jnp.float32),
        ],
    )
    def k(h_hbm, src_hbm, dst_hbm, w_hbm, z_hbm, out_hbm,
          acc_sh, src_v, w_v, dst_v, msg_v):
        c = lax.axis_index("c")
        s = lax.axis_index("s")
        wid = s * _NC + c
        base = wid * epw
        pltpu.sync_copy(z_hbm.at[pl.ds(s * rows, rows)],
                        acc_sh.at[pl.ds(s * rows, rows)])
        pltpu.sync_copy(src_hbm.at[pl.ds(base, epw)], src_v)
        pltpu.sync_copy(w_hbm.at[pl.ds(base, epw)], w_v)
        plsc.subcore_barrier()

        @pl.loop(0, nch)
        def _(ci):
            off = ci * kchunk
            pltpu.sync_copy(dst_hbm.at[pl.ds(base + off, kchunk)], dst_v)
            pltpu.sync_copy(h_hbm.at[src_v.at[pl.ds(off, kchunk)]], msg_v)

            @pl.loop(0, kchunk)
            def _(kk):
                wspl = plsc.load_gather(
                    w_v, [jnp.full((_L,), off + kk, jnp.int32)])
                for j in range(_D // _L):
                    msg_v[kk, pl.ds(j * _L, _L)] = (
                        msg_v[kk, pl.ds(j * _L, _L)] * wspl)

            pltpu.sync_copy(msg_v, acc_sh.at[dst_v], add=True)

        plsc.subcore_barrier()
        pltpu.sync_copy(acc_sh.at[pl.ds(s * rows, rows)],
                        out_hbm.at[c, pl.ds(s * rows, rows)])

    return k(h, src, dst, w, zeros_nd)


def _tc_matmul(x, wmat):
    def body(x_ref, w_ref, o_ref):
        o_ref[...] = jnp.dot(x_ref[...], w_ref[...],
                             preferred_element_type=jnp.float32)

    return pl.pallas_call(
        body,
        out_shape=jax.ShapeDtypeStruct((x.shape[0], wmat.shape[1]),
                                       jnp.float32),
    )(x, wmat)


def _tc_prep(deg_parts, h1):
    n = h1.shape[0]

    def body(dp_ref, h_ref, disb_ref, hs_ref):
        deg = dp_ref[0, :n, 0:1] + dp_ref[1, :n, 0:1] + 1.0
        dis = jnp.where(deg > 0, lax.rsqrt(jnp.maximum(deg, 1e-12)), 0.0)
        disb = jnp.broadcast_to(dis, (n, _D))
        disb_ref[...] = disb
        hs_ref[...] = h_ref[...] * disb

    return pl.pallas_call(
        body,
        out_shape=[jax.ShapeDtypeStruct((n, _D), jnp.float32),
                   jax.ShapeDtypeStruct((n, _D), jnp.float32)],
    )(deg_parts, h1)


def _tc_combine(parts, h_prev, disb, b_row, w_next):
    n = h_prev.shape[0]

    def body(p_ref, h_ref, d_ref, b_ref, w_ref, hn_ref, hns_ref):
        dd = d_ref[...]
        a = (dd * (p_ref[0, :n] + p_ref[1, :n])
             + dd * dd * h_ref[...] + b_ref[...])
        r = jnp.maximum(a, 0.0)
        hn = jnp.dot(r, w_ref[...], preferred_element_type=jnp.float32)
        hn_ref[...] = hn
        hns_ref[...] = hn * dd

    return pl.pallas_call(
        body,
        out_shape=[jax.ShapeDtypeStruct((n, _D), jnp.float32),
                   jax.ShapeDtypeStruct((n, _D), jnp.float32)],
    )(parts, h_prev, disb, b_row, w_next)


def _tc_final(parts, h_prev, disb, b_row, batch_row, wlin, blin_row, g):
    n = h_prev.shape[0]
    ncls = wlin.shape[1]

    def body(p_ref, h_ref, d_ref, b_ref, bat_ref, wl_ref, bl_ref, o_ref):
        dd = d_ref[...]
        a = (dd * (p_ref[0, :n] + p_ref[1, :n])
             + dd * dd * h_ref[...] + b_ref[...])
        r = jnp.maximum(a, 0.0)
        gids = lax.broadcasted_iota(jnp.int32, (g, n), 0)
        oh = (bat_ref[...] == gids).astype(jnp.float32)
        sums = jnp.dot(oh, r, preferred_element_type=jnp.float32)
        cnt = jnp.sum(oh, axis=1, keepdims=True)
        pooled = sums / jnp.maximum(cnt, 1.0)
        o_ref[...] = (jnp.dot(pooled, wl_ref[...],
                              preferred_element_type=jnp.float32)
                      + bl_ref[...])

    return pl.pallas_call(
        body,
        out_shape=jax.ShapeDtypeStruct((g, ncls), jnp.float32),
    )(parts, h_prev, disb, b_row, batch_row, wlin, blin_row)


def kernel(x, edge_index, edge_attr, batch, W1, b1, W2, b2, W3, b3,
           Wlin, blin):
    n, d = x.shape
    assert d == _D
    g = 64
    src = edge_index[0].astype(jnp.int32)
    dst = edge_index[1].astype(jnp.int32)
    w = edge_attr.astype(jnp.float32)
    npad = -(-n // (_NS * 8)) * (_NS * 8)
    zeros_nd = jnp.zeros((npad, _D), jnp.float32)

    deg_parts = _sc_degree(dst, w, zeros_nd, npad)
    h1 = _tc_matmul(x, W1)
    disb, h1s = _tc_prep(deg_parts, h1)

    p1 = _sc_aggregate(h1s, src, dst, w, zeros_nd)
    h2, h2s = _tc_combine(p1, h1, disb, b1.reshape(1, -1), W2)
    p2 = _sc_aggregate(h2s, src, dst, w, zeros_nd)
    h3, h3s = _tc_combine(p2, h2, disb, b2.reshape(1, -1), W3)
    p3 = _sc_aggregate(h3s, src, dst, w, zeros_nd)

    return _tc_final(p3, h3, disb, b3.reshape(1, -1),
                     batch.reshape(1, -1).astype(jnp.int32),
                     Wlin, blin.reshape(1, -1), g)


# speedup vs baseline: 13.5012x; 1.4932x over previous
import dataclasses
import functools

import jax
import jax.numpy as jnp
from jax import lax
from jax.experimental import pallas as pl
from jax.experimental.pallas import tpu as pltpu
from jax.experimental.pallas import tpu_sc as plsc

_NC = 2
_NS = 16
_L = 16
_D = 128


def _pick_chunk(epw):
    for k in range(128, 7, -8):
        if epw % k == 0:
            return k
    raise ValueError(f"no valid chunk size for {epw} edges per worker")


def _sc_mesh():
    return plsc.VectorSubcoreMesh(core_axis_name="c", subcore_axis_name="s")


def _sc_compiler_params():
    cp = pltpu.CompilerParams()
    if "needs_layout_passes" in pltpu.CompilerParams.__dataclass_fields__:
        cp = dataclasses.replace(cp, needs_layout_passes=False)
    return cp


def _sc_degree(dst, w, zeros_nd, npad):
    e = dst.shape[0]
    nw = _NC * _NS
    epw = e // nw
    kchunk = _pick_chunk(epw)
    nch = epw // kchunk
    rows = npad // _NS

    @functools.partial(
        pl.kernel,
        mesh=_sc_mesh(),
        compiler_params=_sc_compiler_params(),
        out_type=jax.ShapeDtypeStruct((_NC, npad, _D), jnp.float32),
        scratch_types=[
            pltpu.VMEM_SHARED((npad, _D), jnp.float32),
            pltpu.VMEM((epw,), jnp.float32),
            pltpu.VMEM((kchunk,), jnp.int32),
            pltpu.VMEM((kchunk, _D), jnp.float32),
        ],
    )
    def k(dst_hbm, w_hbm, z_hbm, out_hbm, deg_sh, w_v, dst_v, wrow_v):
        c = lax.axis_index("c")
        s = lax.axis_index("s")
        wid = s * _NC + c
        base = wid * epw
        pltpu.sync_copy(z_hbm.at[pl.ds(s * rows, rows)],
                        deg_sh.at[pl.ds(s * rows, rows)])
        pltpu.sync_copy(w_hbm.at[pl.ds(base, epw)], w_v)
        plsc.subcore_barrier()

        @pl.loop(0, nch)
        def _(ci):
            off = ci * kchunk
            pltpu.sync_copy(dst_hbm.at[pl.ds(base + off, kchunk)], dst_v)

            @pl.loop(0, kchunk)
            def _(kk):
                wspl = plsc.load_gather(
                    w_v, [jnp.full((_L,), off + kk, jnp.int32)])
                for j in range(_D // _L):
                    wrow_v[kk, pl.ds(j * _L, _L)] = wspl

            pltpu.sync_copy(wrow_v, deg_sh.at[dst_v], add=True)

        plsc.subcore_barrier()
        pltpu.sync_copy(deg_sh.at[pl.ds(s * rows, rows)],
                        out_hbm.at[c, pl.ds(s * rows, rows)])

    return k(dst, w, zeros_nd)


def _sc_aggregate(h, src, dst, w, zeros_nd):
    n = h.shape[0]
    e = src.shape[0]
    npad = zeros_nd.shape[0]
    nw = _NC * _NS
    epw = e // nw
    kchunk = None
    for cand in range(128, 7, -8):
        if epw % cand == 0 and (epw // cand) % 2 == 0:
            kchunk = cand
            break
    assert kchunk is not None
    nch = epw // kchunk
    rows = npad // _NS

    @functools.partial(
        pl.kernel,
        mesh=_sc_mesh(),
        compiler_params=_sc_compiler_params(),
        out_type=jax.ShapeDtypeStruct((_NC, npad, _D), jnp.float32),
        scratch_types=[
            pltpu.VMEM_SHARED((npad, _D), jnp.float32),
            pltpu.VMEM((epw,), jnp.int32),
            pltpu.VMEM((epw,), jnp.float32),
            pltpu.VMEM((kchunk,), jnp.int32),
            pltpu.VMEM((kchunk,), jnp.int32),
            pltpu.VMEM((kchunk, _D), jnp.float32),
            pltpu.VMEM((kchunk, _D), jnp.float32),
            pltpu.SemaphoreType.DMA,
            pltpu.SemaphoreType.DMA,
            pltpu.SemaphoreType.DMA,
            pltpu.SemaphoreType.DMA,
            pltpu.SemaphoreType.DMA,
            pltpu.SemaphoreType.DMA,
        ],
    )
    def k(h_hbm, src_hbm, dst_hbm, w_hbm, z_hbm, out_hbm,
          acc_sh, src_v, w_v, dst0, dst1, msg0, msg1,
          g0, g1, d0, d1, s0, s1):
        c = lax.axis_index("c")
        s = lax.axis_index("s")
        wid = s * _NC + c
        base = wid * epw
        pltpu.sync_copy(z_hbm.at[pl.ds(s * rows, rows)],
                        acc_sh.at[pl.ds(s * rows, rows)])
        pltpu.sync_copy(src_hbm.at[pl.ds(base, epw)], src_v)
        pltpu.sync_copy(w_hbm.at[pl.ds(base, epw)], w_v)
        plsc.subcore_barrier()

        def issue(ci, msgb, dstb, gs, ds):
            pltpu.async_copy(
                dst_hbm.at[pl.ds(base + ci * kchunk, kchunk)], dstb, ds)
            pltpu.async_copy(
                h_hbm.at[src_v.at[pl.ds(ci * kchunk, kchunk)]], msgb, gs)

        def process(ci, msgb, dstb, gs, ds, ss):
            pltpu.make_async_copy(
                h_hbm.at[src_v.at[pl.ds(ci * kchunk, kchunk)]], msgb,
                gs).wait()
            off = ci * kchunk

            @pl.loop(0, kchunk)
            def _(kk):
                wspl = plsc.load_gather(
                    w_v, [jnp.full((_L,), off + kk, jnp.int32)])
                for j in range(_D // _L):
                    msgb[kk, pl.ds(j * _L, _L)] = (
                        msgb[kk, pl.ds(j * _L, _L)] * wspl)

            pltpu.make_async_copy(
                dst_hbm.at[pl.ds(base + ci * kchunk, kchunk)], dstb,
                ds).wait()
            pltpu.async_copy(msgb, acc_sh.at[dstb], ss, add=True)

        def wait_scatter(msgb, dstb, ss):
            pltpu.make_async_copy(msgb, acc_sh.at[dstb], ss).wait()

        issue(0, msg0, dst0, g0, d0)
        issue(1, msg1, dst1, g1, d1)

        @pl.loop(0, nch // 2 - 1)
        def _(i):
            c0 = 2 * i
            process(c0, msg0, dst0, g0, d0, s0)
            wait_scatter(msg0, dst0, s0)
            issue(c0 + 2, msg0, dst0, g0, d0)
            process(c0 + 1, msg1, dst1, g1, d1, s1)
            wait_scatter(msg1, dst1, s1)
            issue(c0 + 3, msg1, dst1, g1, d1)

        process(nch - 2, msg0, dst0, g0, d0, s0)
        wait_scatter(msg0, dst0, s0)
        process(nch - 1, msg1, dst1, g1, d1, s1)
        wait_scatter(msg1, dst1, s1)

        plsc.subcore_barrier()
        pltpu.sync_copy(acc_sh.at[pl.ds(s * rows, rows)],
                        out_hbm.at[c, pl.ds(s * rows, rows)])

    return k(h, src, dst, w, zeros_nd)


def _tc_matmul(x, wmat):
    def body(x_ref, w_ref, o_ref):
        o_ref[...] = jnp.dot(x_ref[...], w_ref[...],
                             preferred_element_type=jnp.float32)

    return pl.pallas_call(
        body,
        out_shape=jax.ShapeDtypeStruct((x.shape[0], wmat.shape[1]),
                                       jnp.float32),
    )(x, wmat)


def _tc_prep(deg_parts, h1):
    n = h1.shape[0]

    def body(dp_ref, h_ref, disb_ref, hs_ref):
        deg = dp_ref[0, :n, 0:1] + dp_ref[1, :n, 0:1] + 1.0
        dis = jnp.where(deg > 0, lax.rsqrt(jnp.maximum(deg, 1e-12)), 0.0)
        disb = jnp.broadcast_to(dis, (n, _D))
        disb_ref[...] = disb
        hs_ref[...] = h_ref[...] * disb

    return pl.pallas_call(
        body,
        out_shape=[jax.ShapeDtypeStruct((n, _D), jnp.float32),
                   jax.ShapeDtypeStruct((n, _D), jnp.float32)],
    )(deg_parts, h1)


def _tc_combine(parts, h_prev, disb, b_row, w_next):
    n = h_prev.shape[0]

    def body(p_ref, h_ref, d_ref, b_ref, w_ref, hn_ref, hns_ref):
        dd = d_ref[...]
        a = (dd * (p_ref[0, :n] + p_ref[1, :n])
             + dd * dd * h_ref[...] + b_ref[...])
        r = jnp.maximum(a, 0.0)
        hn = jnp.dot(r, w_ref[...], preferred_element_type=jnp.float32)
        hn_ref[...] = hn
        hns_ref[...] = hn * dd

    return pl.pallas_call(
        body,
        out_shape=[jax.ShapeDtypeStruct((n, _D), jnp.float32),
                   jax.ShapeDtypeStruct((n, _D), jnp.float32)],
    )(parts, h_prev, disb, b_row, w_next)


def _tc_final(parts, h_prev, disb, b_row, batch_row, wlin, blin_row, g):
    n = h_prev.shape[0]
    ncls = wlin.shape[1]

    def body(p_ref, h_ref, d_ref, b_ref, bat_ref, wl_ref, bl_ref, o_ref):
        dd = d_ref[...]
        a = (dd * (p_ref[0, :n] + p_ref[1, :n])
             + dd * dd * h_ref[...] + b_ref[...])
        r = jnp.maximum(a, 0.0)
        gids = lax.broadcasted_iota(jnp.int32, (g, n), 0)
        oh = (bat_ref[...] == gids).astype(jnp.float32)
        sums = jnp.dot(oh, r, preferred_element_type=jnp.float32)
        cnt = jnp.sum(oh, axis=1, keepdims=True)
        pooled = sums / jnp.maximum(cnt, 1.0)
        o_ref[...] = (jnp.dot(pooled, wl_ref[...],
                              preferred_element_type=jnp.float32)
                      + bl_ref[...])

    return pl.pallas_call(
        body,
        out_shape=jax.ShapeDtypeStruct((g, ncls), jnp.float32),
    )(parts, h_prev, disb, b_row, batch_row, wlin, blin_row)


def kernel(x, edge_index, edge_attr, batch, W1, b1, W2, b2, W3, b3,
           Wlin, blin):
    n, d = x.shape
    assert d == _D
    g = 64
    src = edge_index[0].astype(jnp.int32)
    dst = edge_index[1].astype(jnp.int32)
    w = edge_attr.astype(jnp.float32)
    npad = -(-n // (_NS * 8)) * (_NS * 8)
    zeros_nd = jnp.zeros((npad, _D), jnp.float32)

    deg_parts = _sc_degree(dst, w, zeros_nd, npad)
    h1 = _tc_matmul(x, W1)
    disb, h1s = _tc_prep(deg_parts, h1)

    p1 = _sc_aggregate(h1s, src, dst, w, zeros_nd)
    h2, h2s = _tc_combine(p1, h1, disb, b1.reshape(1, -1), W2)
    p2 = _sc_aggregate(h2s, src, dst, w, zeros_nd)
    h3, h3s = _tc_combine(p2, h2, disb, b2.reshape(1, -1), W3)
    p3 = _sc_aggregate(h3s, src, dst, w, zeros_nd)

    return _tc_final(p3, h3, disb, b3.reshape(1, -1),
                     batch.reshape(1, -1).astype(jnp.int32),
                     Wlin, blin.reshape(1, -1), g)


# speedup vs baseline: 14.2140x; 1.0528x over previous
import dataclasses
import functools

import jax
import jax.numpy as jnp
from jax import lax
from jax.experimental import pallas as pl
from jax.experimental.pallas import tpu as pltpu
from jax.experimental.pallas import tpu_sc as plsc

_NC = 2
_NS = 16
_L = 16
_D = 128


def _pick_chunk(epw):
    for k in range(128, 7, -8):
        if epw % k == 0:
            return k
    raise ValueError(f"no valid chunk size for {epw} edges per worker")


def _sc_mesh():
    return plsc.VectorSubcoreMesh(core_axis_name="c", subcore_axis_name="s")


def _sc_compiler_params():
    cp = pltpu.CompilerParams()
    if "needs_layout_passes" in pltpu.CompilerParams.__dataclass_fields__:
        cp = dataclasses.replace(cp, needs_layout_passes=False)
    return cp


def _sc_degree(dst, w, zeros_nd, npad):
    e = dst.shape[0]
    nw = _NC * _NS
    epw = e // nw
    kchunk = _pick_chunk(epw)
    nch = epw // kchunk
    rows = npad // _NS

    @functools.partial(
        pl.kernel,
        mesh=_sc_mesh(),
        compiler_params=_sc_compiler_params(),
        out_type=jax.ShapeDtypeStruct((_NC, npad, _D), jnp.float32),
        scratch_types=[
            pltpu.VMEM_SHARED((npad, _D), jnp.float32),
            pltpu.VMEM((epw,), jnp.float32),
            pltpu.VMEM((kchunk,), jnp.int32),
            pltpu.VMEM((kchunk, _D), jnp.float32),
        ],
    )
    def k(dst_hbm, w_hbm, z_hbm, out_hbm, deg_sh, w_v, dst_v, wrow_v):
        c = lax.axis_index("c")
        s = lax.axis_index("s")
        wid = s * _NC + c
        base = wid * epw
        pltpu.sync_copy(z_hbm.at[pl.ds(s * rows, rows)],
                        deg_sh.at[pl.ds(s * rows, rows)])
        pltpu.sync_copy(w_hbm.at[pl.ds(base, epw)], w_v)
        plsc.subcore_barrier()

        @pl.loop(0, nch)
        def _(ci):
            off = ci * kchunk
            pltpu.sync_copy(dst_hbm.at[pl.ds(base + off, kchunk)], dst_v)

            @pl.loop(0, kchunk)
            def _(kk):
                wspl = plsc.load_gather(
                    w_v, [jnp.full((_L,), off + kk, jnp.int32)])
                for j in range(_D // _L):
                    wrow_v[kk, pl.ds(j * _L, _L)] = wspl

            pltpu.sync_copy(wrow_v, deg_sh.at[dst_v], add=True)

        plsc.subcore_barrier()
        pltpu.sync_copy(deg_sh.at[pl.ds(s * rows, rows)],
                        out_hbm.at[c, pl.ds(s * rows, rows)])

    return k(dst, w, zeros_nd)


def _sc_aggregate(h, src, dst, w, zeros_nd):
    n = h.shape[0]
    e = src.shape[0]
    npad = zeros_nd.shape[0]
    nw = _NC * _NS
    epw = e // nw
    kchunk = None
    for cand in range(128, 7, -8):
        if epw % cand == 0 and (epw // cand) % 2 == 0:
            kchunk = cand
            break
    assert kchunk is not None
    nch = epw // kchunk
    rows = npad // _NS

    @functools.partial(
        pl.kernel,
        mesh=_sc_mesh(),
        compiler_params=_sc_compiler_params(),
        out_type=jax.ShapeDtypeStruct((_NC, npad, _D), jnp.float32),
        scratch_types=[
            pltpu.VMEM_SHARED((npad, _D), jnp.float32),
            pltpu.VMEM((epw,), jnp.int32),
            pltpu.VMEM((epw,), jnp.float32),
            pltpu.VMEM((kchunk,), jnp.int32),
            pltpu.VMEM((kchunk,), jnp.int32),
            pltpu.VMEM((kchunk, _D), jnp.float32),
            pltpu.VMEM((kchunk, _D), jnp.float32),
            pltpu.SemaphoreType.DMA,
            pltpu.SemaphoreType.DMA,
            pltpu.SemaphoreType.DMA,
            pltpu.SemaphoreType.DMA,
            pltpu.SemaphoreType.DMA,
            pltpu.SemaphoreType.DMA,
        ],
    )
    def k(h_hbm, src_hbm, dst_hbm, w_hbm, z_hbm, out_hbm,
          acc_sh, src_v, w_v, dst0, dst1, msg0, msg1,
          g0, g1, d0, d1, s0, s1):
        c = lax.axis_index("c")
        s = lax.axis_index("s")
        wid = s * _NC + c
        base = wid * epw
        pltpu.sync_copy(z_hbm.at[pl.ds(s * rows, rows)],
                        acc_sh.at[pl.ds(s * rows, rows)])
        pltpu.sync_copy(src_hbm.at[pl.ds(base, epw)], src_v)
        pltpu.sync_copy(w_hbm.at[pl.ds(base, epw)], w_v)
        plsc.subcore_barrier()

        def issue(ci, msgb, dstb, gs, ds):
            pltpu.async_copy(
                dst_hbm.at[pl.ds(base + ci * kchunk, kchunk)], dstb, ds)
            pltpu.async_copy(
                h_hbm.at[src_v.at[pl.ds(ci * kchunk, kchunk)]], msgb, gs)

        def process(ci, msgb, dstb, gs, ds, ss):
            pltpu.make_async_copy(
                h_hbm.at[src_v.at[pl.ds(ci * kchunk, kchunk)]], msgb,
                gs).wait()
            off = ci * kchunk

            @pl.loop(0, kchunk, step=2)
            def _(kk):
                wspl0 = plsc.load_gather(
                    w_v, [jnp.full((_L,), off + kk, jnp.int32)])
                wspl1 = plsc.load_gather(
                    w_v, [jnp.full((_L,), off + kk + 1, jnp.int32)])
                nj = _D // _L
                r0 = [msgb[kk, pl.ds(j * _L, _L)] for j in range(nj)]
                r1 = [msgb[kk + 1, pl.ds(j * _L, _L)] for j in range(nj)]
                for j in range(nj):
                    msgb[kk, pl.ds(j * _L, _L)] = r0[j] * wspl0
                    msgb[kk + 1, pl.ds(j * _L, _L)] = r1[j] * wspl1

            pltpu.make_async_copy(
                dst_hbm.at[pl.ds(base + ci * kchunk, kchunk)], dstb,
                ds).wait()
            pltpu.async_copy(msgb, acc_sh.at[dstb], ss, add=True)

        def wait_scatter(msgb, dstb, ss):
            pltpu.make_async_copy(msgb, acc_sh.at[dstb], ss).wait()

        issue(0, msg0, dst0, g0, d0)
        issue(1, msg1, dst1, g1, d1)

        @pl.loop(0, nch // 2 - 1)
        def _(i):
            c0 = 2 * i
            process(c0, msg0, dst0, g0, d0, s0)
            wait_scatter(msg0, dst0, s0)
            issue(c0 + 2, msg0, dst0, g0, d0)
            process(c0 + 1, msg1, dst1, g1, d1, s1)
            wait_scatter(msg1, dst1, s1)
            issue(c0 + 3, msg1, dst1, g1, d1)

        process(nch - 2, msg0, dst0, g0, d0, s0)
        wait_scatter(msg0, dst0, s0)
        process(nch - 1, msg1, dst1, g1, d1, s1)
        wait_scatter(msg1, dst1, s1)

        plsc.subcore_barrier()
        pltpu.sync_copy(acc_sh.at[pl.ds(s * rows, rows)],
                        out_hbm.at[c, pl.ds(s * rows, rows)])

    return k(h, src, dst, w, zeros_nd)


def _tc_matmul(x, wmat):
    def body(x_ref, w_ref, o_ref):
        o_ref[...] = jnp.dot(x_ref[...], w_ref[...],
                             preferred_element_type=jnp.float32)

    return pl.pallas_call(
        body,
        out_shape=jax.ShapeDtypeStruct((x.shape[0], wmat.shape[1]),
                                       jnp.float32),
    )(x, wmat)


def _tc_prep(deg_parts, h1):
    n = h1.shape[0]

    def body(dp_ref, h_ref, disb_ref, hs_ref):
        deg = dp_ref[0, :n, 0:1] + dp_ref[1, :n, 0:1] + 1.0
        dis = jnp.where(deg > 0, lax.rsqrt(jnp.maximum(deg, 1e-12)), 0.0)
        disb = jnp.broadcast_to(dis, (n, _D))
        disb_ref[...] = disb
        hs_ref[...] = h_ref[...] * disb

    return pl.pallas_call(
        body,
        out_shape=[jax.ShapeDtypeStruct((n, _D), jnp.float32),
                   jax.ShapeDtypeStruct((n, _D), jnp.float32)],
    )(deg_parts, h1)


def _tc_combine(parts, h_prev, disb, b_row, w_next):
    n = h_prev.shape[0]

    def body(p_ref, h_ref, d_ref, b_ref, w_ref, hn_ref, hns_ref):
        dd = d_ref[...]
        a = (dd * (p_ref[0, :n] + p_ref[1, :n])
             + dd * dd * h_ref[...] + b_ref[...])
        r = jnp.maximum(a, 0.0)
        hn = jnp.dot(r, w_ref[...], preferred_element_type=jnp.float32)
        hn_ref[...] = hn
        hns_ref[...] = hn * dd

    return pl.pallas_call(
        body,
        out_shape=[jax.ShapeDtypeStruct((n, _D), jnp.float32),
                   jax.ShapeDtypeStruct((n, _D), jnp.float32)],
    )(parts, h_prev, disb, b_row, w_next)


def _tc_final(parts, h_prev, disb, b_row, batch_row, wlin, blin_row, g):
    n = h_prev.shape[0]
    ncls = wlin.shape[1]

    def body(p_ref, h_ref, d_ref, b_ref, bat_ref, wl_ref, bl_ref, o_ref):
        dd = d_ref[...]
        a = (dd * (p_ref[0, :n] + p_ref[1, :n])
             + dd * dd * h_ref[...] + b_ref[...])
        r = jnp.maximum(a, 0.0)
        gids = lax.broadcasted_iota(jnp.int32, (g, n), 0)
        oh = (bat_ref[...] == gids).astype(jnp.float32)
        sums = jnp.dot(oh, r, preferred_element_type=jnp.float32)
        cnt = jnp.sum(oh, axis=1, keepdims=True)
        pooled = sums / jnp.maximum(cnt, 1.0)
        o_ref[...] = (jnp.dot(pooled, wl_ref[...],
                              preferred_element_type=jnp.float32)
                      + bl_ref[...])

    return pl.pallas_call(
        body,
        out_shape=jax.ShapeDtypeStruct((g, ncls), jnp.float32),
    )(parts, h_prev, disb, b_row, batch_row, wlin, blin_row)


def kernel(x, edge_index, edge_attr, batch, W1, b1, W2, b2, W3, b3,
           Wlin, blin):
    n, d = x.shape
    assert d == _D
    g = 64
    src = edge_index[0].astype(jnp.int32)
    dst = edge_index[1].astype(jnp.int32)
    w = edge_attr.astype(jnp.float32)
    npad = -(-n // (_NS * 8)) * (_NS * 8)
    zeros_nd = jnp.zeros((npad, _D), jnp.float32)

    deg_parts = _sc_degree(dst, w, zeros_nd, npad)
    h1 = _tc_matmul(x, W1)
    disb, h1s = _tc_prep(deg_parts, h1)

    p1 = _sc_aggregate(h1s, src, dst, w, zeros_nd)
    h2, h2s = _tc_combine(p1, h1, disb, b1.reshape(1, -1), W2)
    p2 = _sc_aggregate(h2s, src, dst, w, zeros_nd)
    h3, h3s = _tc_combine(p2, h2, disb, b2.reshape(1, -1), W3)
    p3 = _sc_aggregate(h3s, src, dst, w, zeros_nd)

    return _tc_final(p3, h3, disb, b3.reshape(1, -1),
                     batch.reshape(1, -1).astype(jnp.int32),
                     Wlin, blin.reshape(1, -1), g)


# speedup vs baseline: 16.5953x; 1.1675x over previous
import dataclasses
import functools

import jax
import jax.numpy as jnp
from jax import lax
from jax.experimental import pallas as pl
from jax.experimental.pallas import tpu as pltpu
from jax.experimental.pallas import tpu_sc as plsc

_NC = 2
_NS = 16
_L = 16
_D = 128


def _pick_chunk(epw):
    for k in range(128, 7, -8):
        if epw % k == 0:
            return k
    raise ValueError(f"no valid chunk size for {epw} edges per worker")


def _sc_mesh():
    return plsc.VectorSubcoreMesh(core_axis_name="c", subcore_axis_name="s")


def _sc_compiler_params():
    cp = pltpu.CompilerParams()
    if "needs_layout_passes" in pltpu.CompilerParams.__dataclass_fields__:
        cp = dataclasses.replace(cp, needs_layout_passes=False)
    return cp


def _sc_degree(dst, w, zeros_nd, npad):
    e = dst.shape[0]
    nw = _NC * _NS
    epw = e // nw
    kchunk = _pick_chunk(epw)
    nch = epw // kchunk
    rows = npad // _NS

    @functools.partial(
        pl.kernel,
        mesh=_sc_mesh(),
        compiler_params=_sc_compiler_params(),
        out_type=jax.ShapeDtypeStruct((_NC, npad, _D), jnp.float32),
        scratch_types=[
            pltpu.VMEM_SHARED((npad, _D), jnp.float32),
            pltpu.VMEM((epw,), jnp.float32),
            pltpu.VMEM((kchunk,), jnp.int32),
            pltpu.VMEM((kchunk, _D), jnp.float32),
        ],
    )
    def k(dst_hbm, w_hbm, z_hbm, out_hbm, deg_sh, w_v, dst_v, wrow_v):
        c = lax.axis_index("c")
        s = lax.axis_index("s")
        wid = s * _NC + c
        base = wid * epw
        pltpu.sync_copy(z_hbm.at[pl.ds(s * rows, rows)],
                        deg_sh.at[pl.ds(s * rows, rows)])
        pltpu.sync_copy(w_hbm.at[pl.ds(base, epw)], w_v)
        plsc.subcore_barrier()

        @pl.loop(0, nch)
        def _(ci):
            off = ci * kchunk
            pltpu.sync_copy(dst_hbm.at[pl.ds(base + off, kchunk)], dst_v)

            @pl.loop(0, kchunk)
            def _(kk):
                wspl = plsc.load_gather(
                    w_v, [jnp.full((_L,), off + kk, jnp.int32)])
                for j in range(_D // _L):
                    wrow_v[kk, pl.ds(j * _L, _L)] = wspl

            pltpu.sync_copy(wrow_v, deg_sh.at[dst_v], add=True)

        plsc.subcore_barrier()
        pltpu.sync_copy(deg_sh.at[pl.ds(s * rows, rows)],
                        out_hbm.at[c, pl.ds(s * rows, rows)])

    return k(dst, w, zeros_nd)


def _sc_aggregate(h, src, dst, w, zeros_nd):
    n = h.shape[0]
    e = src.shape[0]
    npad = zeros_nd.shape[0]
    nw = _NC * _NS
    epw = e // nw
    kchunk = None
    for cand in range(128, 7, -8):
        if epw % cand == 0 and epw // cand >= 8 and (epw // cand) % 2 == 1:
            kchunk = cand
            break
    assert kchunk is not None
    nch = epw // kchunk
    rows = npad // _NS
    nbuf = 2
    rem = nch % nbuf
    n_main = (nch - nbuf - rem) // nbuf
    n_tail = nch - n_main * nbuf - nbuf

    @functools.partial(
        pl.kernel,
        mesh=_sc_mesh(),
        compiler_params=_sc_compiler_params(),
        out_type=jax.ShapeDtypeStruct((_NC, npad, _D), jnp.float32),
        scratch_types=[
            pltpu.VMEM_SHARED((npad, _D), jnp.float32),
            pltpu.VMEM((epw,), jnp.int32),
            pltpu.VMEM((epw,), jnp.float32),
        ] + [pltpu.VMEM((kchunk,), jnp.int32) for _ in range(nbuf)]
          + [pltpu.VMEM((kchunk, _D), jnp.float32) for _ in range(nbuf)]
          + [pltpu.SemaphoreType.DMA for _ in range(3 * nbuf)],
    )
    def k(h_hbm, src_hbm, dst_hbm, w_hbm, z_hbm, out_hbm,
          acc_sh, src_v, w_v, *bufs):
        dsts = bufs[0:nbuf]
        msgs = bufs[nbuf:2 * nbuf]
        gsems = bufs[2 * nbuf:3 * nbuf]
        dsems = bufs[3 * nbuf:4 * nbuf]
        ssems = bufs[4 * nbuf:5 * nbuf]
        c = lax.axis_index("c")
        s = lax.axis_index("s")
        wid = s * _NC + c
        base = wid * epw
        pltpu.sync_copy(z_hbm.at[pl.ds(s * rows, rows)],
                        acc_sh.at[pl.ds(s * rows, rows)])
        pltpu.sync_copy(src_hbm.at[pl.ds(base, epw)], src_v)
        pltpu.sync_copy(w_hbm.at[pl.ds(base, epw)], w_v)
        plsc.subcore_barrier()

        def issue(ci, b):
            pltpu.async_copy(
                dst_hbm.at[pl.ds(base + ci * kchunk, kchunk)],
                dsts[b], dsems[b])
            pltpu.async_copy(
                h_hbm.at[src_v.at[pl.ds(ci * kchunk, kchunk)]],
                msgs[b], gsems[b])

        def process(ci, b):
            pltpu.make_async_copy(
                h_hbm.at[src_v.at[pl.ds(ci * kchunk, kchunk)]],
                msgs[b], gsems[b]).wait()
            off = ci * kchunk
            msgb = msgs[b]

            @pl.loop(0, kchunk, step=2)
            def _(kk):
                wspl0 = plsc.load_gather(
                    w_v, [jnp.full((_L,), off + kk, jnp.int32)])
                wspl1 = plsc.load_gather(
                    w_v, [jnp.full((_L,), off + kk + 1, jnp.int32)])
                nj = _D // _L
                r0 = [msgb[kk, pl.ds(j * _L, _L)] for j in range(nj)]
                r1 = [msgb[kk + 1, pl.ds(j * _L, _L)] for j in range(nj)]
                for j in range(nj):
                    msgb[kk, pl.ds(j * _L, _L)] = r0[j] * wspl0
                    msgb[kk + 1, pl.ds(j * _L, _L)] = r1[j] * wspl1

            pltpu.make_async_copy(
                dst_hbm.at[pl.ds(base + ci * kchunk, kchunk)],
                dsts[b], dsems[b]).wait()
            pltpu.async_copy(msgs[b], acc_sh.at[dsts[b]], ssems[b],
                             add=True)

        def wait_scatter(b):
            pltpu.make_async_copy(msgs[b], acc_sh.at[dsts[b]],
                                  ssems[b]).wait()

        for b in range(nbuf):
            issue(b, b)

        @pl.loop(0, n_main)
        def _(i):
            c0 = i * nbuf
            for b in range(nbuf):
                process(c0 + b, b)
                wait_scatter(b)
                issue(c0 + nbuf + b, b)

        tail_base = n_main * nbuf
        for idx in range(nbuf + n_tail):
            ci = tail_base + idx
            b = idx % nbuf
            process(ci, b)
            wait_scatter(b)
            nxt = ci + nbuf
            if nxt < nch:
                issue(nxt, b)

        plsc.subcore_barrier()
        pltpu.sync_copy(acc_sh.at[pl.ds(s * rows, rows)],
                        out_hbm.at[c, pl.ds(s * rows, rows)])

    return k(h, src, dst, w, zeros_nd)


def _tc_matmul(x, wmat):
    def body(x_ref, w_ref, o_ref):
        o_ref[...] = jnp.dot(x_ref[...], w_ref[...],
                             preferred_element_type=jnp.float32)

    return pl.pallas_call(
        body,
        out_shape=jax.ShapeDtypeStruct((x.shape[0], wmat.shape[1]),
                                       jnp.float32),
    )(x, wmat)


def _tc_prep(deg_parts, h1):
    n = h1.shape[0]

    def body(dp_ref, h_ref, disb_ref, hs_ref):
        deg = dp_ref[0, :n, 0:1] + dp_ref[1, :n, 0:1] + 1.0
        dis = jnp.where(deg > 0, lax.rsqrt(jnp.maximum(deg, 1e-12)), 0.0)
        disb = jnp.broadcast_to(dis, (n, _D))
        disb_ref[...] = disb
        hs_ref[...] = h_ref[...] * disb

    return pl.pallas_call(
        body,
        out_shape=[jax.ShapeDtypeStruct((n, _D), jnp.float32),
                   jax.ShapeDtypeStruct((n, _D), jnp.float32)],
    )(deg_parts, h1)


def _tc_combine(parts, h_prev, disb, b_row, w_next):
    n = h_prev.shape[0]

    def body(p_ref, h_ref, d_ref, b_ref, w_ref, hn_ref, hns_ref):
        dd = d_ref[...]
        a = (dd * (p_ref[0, :n] + p_ref[1, :n])
             + dd * dd * h_ref[...] + b_ref[...])
        r = jnp.maximum(a, 0.0)
        hn = jnp.dot(r, w_ref[...], preferred_element_type=jnp.float32)
        hn_ref[...] = hn
        hns_ref[...] = hn * dd

    return pl.pallas_call(
        body,
        out_shape=[jax.ShapeDtypeStruct((n, _D), jnp.float32),
                   jax.ShapeDtypeStruct((n, _D), jnp.float32)],
    )(parts, h_prev, disb, b_row, w_next)


def _tc_final(parts, h_prev, disb, b_row, batch_row, wlin, blin_row, g):
    n = h_prev.shape[0]
    ncls = wlin.shape[1]

    def body(p_ref, h_ref, d_ref, b_ref, bat_ref, wl_ref, bl_ref, o_ref):
        dd = d_ref[...]
        a = (dd * (p_ref[0, :n] + p_ref[1, :n])
             + dd * dd * h_ref[...] + b_ref[...])
        r = jnp.maximum(a, 0.0)
        gids = lax.broadcasted_iota(jnp.int32, (g, n), 0)
        oh = (bat_ref[...] == gids).astype(jnp.float32)
        sums = jnp.dot(oh, r, preferred_element_type=jnp.float32)
        cnt = jnp.sum(oh, axis=1, keepdims=True)
        pooled = sums / jnp.maximum(cnt, 1.0)
        o_ref[...] = (jnp.dot(pooled, wl_ref[...],
                              preferred_element_type=jnp.float32)
                      + bl_ref[...])

    return pl.pallas_call(
        body,
        out_shape=jax.ShapeDtypeStruct((g, ncls), jnp.float32),
    )(parts, h_prev, disb, b_row, batch_row, wlin, blin_row)


def kernel(x, edge_index, edge_attr, batch, W1, b1, W2, b2, W3, b3,
           Wlin, blin):
    n, d = x.shape
    assert d == _D
    g = 64
    src = edge_index[0].astype(jnp.int32)
    dst = edge_index[1].astype(jnp.int32)
    w = edge_attr.astype(jnp.float32)
    npad = -(-n // (_NS * 8)) * (_NS * 8)
    zeros_nd = jnp.zeros((npad, _D), jnp.float32)

    deg_parts = _sc_degree(dst, w, zeros_nd, npad)
    h1 = _tc_matmul(x, W1)
    disb, h1s = _tc_prep(deg_parts, h1)

    p1 = _sc_aggregate(h1s, src, dst, w, zeros_nd)
    h2, h2s = _tc_combine(p1, h1, disb, b1.reshape(1, -1), W2)
    p2 = _sc_aggregate(h2s, src, dst, w, zeros_nd)
    h3, h3s = _tc_combine(p2, h2, disb, b2.reshape(1, -1), W3)
    p3 = _sc_aggregate(h3s, src, dst, w, zeros_nd)

    return _tc_final(p3, h3, disb, b3.reshape(1, -1),
                     batch.reshape(1, -1).astype(jnp.int32),
                     Wlin, blin.reshape(1, -1), g)


# speedup vs baseline: 19.2715x; 1.1613x over previous
import dataclasses
import functools

import jax
import jax.numpy as jnp
from jax import lax
from jax.experimental import pallas as pl
from jax.experimental.pallas import tpu as pltpu
from jax.experimental.pallas import tpu_sc as plsc

_NC = 2
_NS = 16
_L = 16
_D = 128


def _pick_chunk(epw):
    for k in range(128, 7, -8):
        if epw % k == 0:
            return k
    raise ValueError(f"no valid chunk size for {epw} edges per worker")


def _sc_mesh():
    return plsc.VectorSubcoreMesh(core_axis_name="c", subcore_axis_name="s")


def _sc_compiler_params():
    cp = pltpu.CompilerParams()
    if "needs_layout_passes" in pltpu.CompilerParams.__dataclass_fields__:
        cp = dataclasses.replace(cp, needs_layout_passes=False)
    return cp


def _sc_degree(dst, w, zeros_nd, npad):
    e = dst.shape[0]
    nw = _NC * _NS
    epw = e // nw
    kchunk = _pick_chunk(epw)
    nch = epw // kchunk
    rows = npad // _NS
    nbuf = 2
    rem = nch % nbuf
    n_main = (nch - nbuf - rem) // nbuf
    n_tail = nch - n_main * nbuf - nbuf

    @functools.partial(
        pl.kernel,
        mesh=_sc_mesh(),
        compiler_params=_sc_compiler_params(),
        out_type=jax.ShapeDtypeStruct((_NC, npad, _D), jnp.float32),
        scratch_types=[
            pltpu.VMEM_SHARED((npad, _D), jnp.float32),
            pltpu.VMEM((epw,), jnp.float32),
        ] + [pltpu.VMEM((kchunk,), jnp.int32) for _ in range(nbuf)]
          + [pltpu.VMEM((kchunk, _D), jnp.float32) for _ in range(nbuf)]
          + [pltpu.SemaphoreType.DMA for _ in range(2 * nbuf)],
    )
    def k(dst_hbm, w_hbm, z_hbm, out_hbm, deg_sh, w_v, *bufs):
        dsts = bufs[0:nbuf]
        wrows = bufs[nbuf:2 * nbuf]
        dsems = bufs[2 * nbuf:3 * nbuf]
        ssems = bufs[3 * nbuf:4 * nbuf]
        c = lax.axis_index("c")
        s = lax.axis_index("s")
        wid = s * _NC + c
        base = wid * epw
        pltpu.sync_copy(z_hbm.at[pl.ds(s * rows, rows)],
                        deg_sh.at[pl.ds(s * rows, rows)])
        pltpu.sync_copy(w_hbm.at[pl.ds(base, epw)], w_v)
        plsc.subcore_barrier()

        def issue_dst(ci, b):
            pltpu.async_copy(
                dst_hbm.at[pl.ds(base + ci * kchunk, kchunk)],
                dsts[b], dsems[b])

        def process(ci, b):
            off = ci * kchunk
            wrb = wrows[b]

            @pl.loop(0, kchunk, step=2)
            def _(kk):
                wspl0 = plsc.load_gather(
                    w_v, [jnp.full((_L,), off + kk, jnp.int32)])
                wspl1 = plsc.load_gather(
                    w_v, [jnp.full((_L,), off + kk + 1, jnp.int32)])
                wrb[kk, pl.ds(0, _L)] = wspl0
                wrb[kk + 1, pl.ds(0, _L)] = wspl1

            pltpu.make_async_copy(
                dst_hbm.at[pl.ds(base + ci * kchunk, kchunk)],
                dsts[b], dsems[b]).wait()
            pltpu.async_copy(wrows[b], deg_sh.at[dsts[b]], ssems[b],
                             add=True)

        def wait_scatter(b):
            pltpu.make_async_copy(wrows[b], deg_sh.at[dsts[b]],
                                  ssems[b]).wait()

        for b in range(nbuf):
            issue_dst(b, b)

        @pl.loop(0, n_main)
        def _(i):
            c0 = i * nbuf
            for b in range(nbuf):
                process(c0 + b, b)
                wait_scatter(b)
                issue_dst(c0 + nbuf + b, b)

        tail_base = n_main * nbuf
        for idx in range(nbuf + n_tail):
            ci = tail_base + idx
            b = idx % nbuf
            process(ci, b)
            wait_scatter(b)
            nxt = ci + nbuf
            if nxt < nch:
                issue_dst(nxt, b)

        plsc.subcore_barrier()
        pltpu.sync_copy(deg_sh.at[pl.ds(s * rows, rows)],
                        out_hbm.at[c, pl.ds(s * rows, rows)])

    return k(dst, w, zeros_nd)


def _sc_aggregate(h, src, dst, w, zeros_nd):
    n = h.shape[0]
    e = src.shape[0]
    npad = zeros_nd.shape[0]
    nw = _NC * _NS
    epw = e // nw
    kchunk = None
    for cand in range(128, 7, -8):
        if epw % cand == 0 and epw // cand >= 8 and (epw // cand) % 2 == 1:
            kchunk = cand
            break
    assert kchunk is not None
    nch = epw // kchunk
    rows = npad // _NS
    nbuf = 2
    rem = nch % nbuf
    n_main = (nch - nbuf - rem) // nbuf
    n_tail = nch - n_main * nbuf - nbuf

    @functools.partial(
        pl.kernel,
        mesh=_sc_mesh(),
        compiler_params=_sc_compiler_params(),
        out_type=jax.ShapeDtypeStruct((_NC, npad, _D), jnp.float32),
        scratch_types=[
            pltpu.VMEM_SHARED((npad, _D), jnp.float32),
            pltpu.VMEM((epw,), jnp.int32),
            pltpu.VMEM((epw,), jnp.float32),
        ] + [pltpu.VMEM((kchunk,), jnp.int32) for _ in range(nbuf)]
          + [pltpu.VMEM((kchunk, _D), jnp.float32) for _ in range(nbuf)]
          + [pltpu.SemaphoreType.DMA for _ in range(3 * nbuf)],
    )
    def k(h_hbm, src_hbm, dst_hbm, w_hbm, z_hbm, out_hbm,
          acc_sh, src_v, w_v, *bufs):
        dsts = bufs[0:nbuf]
        msgs = bufs[nbuf:2 * nbuf]
        gsems = bufs[2 * nbuf:3 * nbuf]
        dsems = bufs[3 * nbuf:4 * nbuf]
        ssems = bufs[4 * nbuf:5 * nbuf]
        c = lax.axis_index("c")
        s = lax.axis_index("s")
        wid = s * _NC + c
        base = wid * epw
        pltpu.sync_copy(z_hbm.at[pl.ds(s * rows, rows)],
                        acc_sh.at[pl.ds(s * rows, rows)])
        pltpu.sync_copy(src_hbm.at[pl.ds(base, epw)], src_v)
        pltpu.sync_copy(w_hbm.at[pl.ds(base, epw)], w_v)
        plsc.subcore_barrier()

        def issue(ci, b):
            pltpu.async_copy(
                dst_hbm.at[pl.ds(base + ci * kchunk, kchunk)],
                dsts[b], dsems[b])
            pltpu.async_copy(
                h_hbm.at[src_v.at[pl.ds(ci * kchunk, kchunk)]],
                msgs[b], gsems[b])

        def process(ci, b):
            pltpu.make_async_copy(
                h_hbm.at[src_v.at[pl.ds(ci * kchunk, kchunk)]],
                msgs[b], gsems[b]).wait()
            off = ci * kchunk
            msgb = msgs[b]

            @pl.loop(0, kchunk, step=2)
            def _(kk):
                wspl0 = plsc.load_gather(
                    w_v, [jnp.full((_L,), off + kk, jnp.int32)])
                wspl1 = plsc.load_gather(
                    w_v, [jnp.full((_L,), off + kk + 1, jnp.int32)])
                nj = _D // _L
                r0 = [msgb[kk, pl.ds(j * _L, _L)] for j in range(nj)]
                r1 = [msgb[kk + 1, pl.ds(j * _L, _L)] for j in range(nj)]
                for j in range(nj):
                    msgb[kk, pl.ds(j * _L, _L)] = r0[j] * wspl0
                    msgb[kk + 1, pl.ds(j * _L, _L)] = r1[j] * wspl1

            pltpu.make_async_copy(
                dst_hbm.at[pl.ds(base + ci * kchunk, kchunk)],
                dsts[b], dsems[b]).wait()
            pltpu.async_copy(msgs[b], acc_sh.at[dsts[b]], ssems[b],
                             add=True)

        def wait_scatter(b):
            pltpu.make_async_copy(msgs[b], acc_sh.at[dsts[b]],
                                  ssems[b]).wait()

        for b in range(nbuf):
            issue(b, b)

        @pl.loop(0, n_main)
        def _(i):
            c0 = i * nbuf
            for b in range(nbuf):
                process(c0 + b, b)
                wait_scatter(b)
                issue(c0 + nbuf + b, b)

        tail_base = n_main * nbuf
        for idx in range(nbuf + n_tail):
            ci = tail_base + idx
            b = idx % nbuf
            process(ci, b)
            wait_scatter(b)
            nxt = ci + nbuf
            if nxt < nch:
                issue(nxt, b)

        plsc.subcore_barrier()
        pltpu.sync_copy(acc_sh.at[pl.ds(s * rows, rows)],
                        out_hbm.at[c, pl.ds(s * rows, rows)])

    return k(h, src, dst, w, zeros_nd)


def _tc_matmul(x, wmat):
    def body(x_ref, w_ref, o_ref):
        o_ref[...] = jnp.dot(x_ref[...], w_ref[...],
                             preferred_element_type=jnp.float32)

    return pl.pallas_call(
        body,
        out_shape=jax.ShapeDtypeStruct((x.shape[0], wmat.shape[1]),
                                       jnp.float32),
    )(x, wmat)


def _tc_prep(deg_parts, h1):
    n = h1.shape[0]

    def body(dp_ref, h_ref, disb_ref, hs_ref):
        deg = dp_ref[0, :n, 0:1] + dp_ref[1, :n, 0:1] + 1.0
        dis = jnp.where(deg > 0, lax.rsqrt(jnp.maximum(deg, 1e-12)), 0.0)
        disb = jnp.broadcast_to(dis, (n, _D))
        disb_ref[...] = disb
        hs_ref[...] = h_ref[...] * disb

    return pl.pallas_call(
        body,
        out_shape=[jax.ShapeDtypeStruct((n, _D), jnp.float32),
                   jax.ShapeDtypeStruct((n, _D), jnp.float32)],
    )(deg_parts, h1)


def _tc_combine(parts, h_prev, disb, b_row, w_next):
    n = h_prev.shape[0]

    def body(p_ref, h_ref, d_ref, b_ref, w_ref, hn_ref, hns_ref):
        dd = d_ref[...]
        a = (dd * (p_ref[0, :n] + p_ref[1, :n])
             + dd * dd * h_ref[...] + b_ref[...])
        r = jnp.maximum(a, 0.0)
        hn = jnp.dot(r, w_ref[...], preferred_element_type=jnp.float32)
        hn_ref[...] = hn
        hns_ref[...] = hn * dd

    return pl.pallas_call(
        body,
        out_shape=[jax.ShapeDtypeStruct((n, _D), jnp.float32),
                   jax.ShapeDtypeStruct((n, _D), jnp.float32)],
    )(parts, h_prev, disb, b_row, w_next)


def _tc_final(parts, h_prev, disb, b_row, batch_row, wlin, blin_row, g):
    n = h_prev.shape[0]
    ncls = wlin.shape[1]

    def body(p_ref, h_ref, d_ref, b_ref, bat_ref, wl_ref, bl_ref, o_ref):
        dd = d_ref[...]
        a = (dd * (p_ref[0, :n] + p_ref[1, :n])
             + dd * dd * h_ref[...] + b_ref[...])
        r = jnp.maximum(a, 0.0)
        gids = lax.broadcasted_iota(jnp.int32, (g, n), 0)
        oh = (bat_ref[...] == gids).astype(jnp.float32)
        sums = jnp.dot(oh, r, preferred_element_type=jnp.float32)
        cnt = jnp.sum(oh, axis=1, keepdims=True)
        pooled = sums / jnp.maximum(cnt, 1.0)
        o_ref[...] = (jnp.dot(pooled, wl_ref[...],
                              preferred_element_type=jnp.float32)
                      + bl_ref[...])

    return pl.pallas_call(
        body,
        out_shape=jax.ShapeDtypeStruct((g, ncls), jnp.float32),
    )(parts, h_prev, disb, b_row, batch_row, wlin, blin_row)


def kernel(x, edge_index, edge_attr, batch, W1, b1, W2, b2, W3, b3,
           Wlin, blin):
    n, d = x.shape
    assert d == _D
    g = 64
    src = edge_index[0].astype(jnp.int32)
    dst = edge_index[1].astype(jnp.int32)
    w = edge_attr.astype(jnp.float32)
    npad = -(-n // (_NS * 8)) * (_NS * 8)
    zeros_nd = jnp.zeros((npad, _D), jnp.float32)

    deg_parts = _sc_degree(dst, w, zeros_nd, npad)
    h1 = _tc_matmul(x, W1)
    disb, h1s = _tc_prep(deg_parts, h1)

    p1 = _sc_aggregate(h1s, src, dst, w, zeros_nd)
    h2, h2s = _tc_combine(p1, h1, disb, b1.reshape(1, -1), W2)
    p2 = _sc_aggregate(h2s, src, dst, w, zeros_nd)
    h3, h3s = _tc_combine(p2, h2, disb, b2.reshape(1, -1), W3)
    p3 = _sc_aggregate(h3s, src, dst, w, zeros_nd)

    return _tc_final(p3, h3, disb, b3.reshape(1, -1),
                     batch.reshape(1, -1).astype(jnp.int32),
                     Wlin, blin.reshape(1, -1), g)
